# Initial kernel scaffold; baseline (speedup 1.0000x reference)
import functools

import jax
import jax.numpy as jnp
from jax import lax
from jax.experimental import pallas as pl
from jax.experimental.pallas import tpu as pltpu
from jax.experimental.pallas import tpu_sc as plsc

_CH = 128
_NSLAB = 32
_RB = 2000


def _cdiv(a, b):
    return (a + b - 1) // b


def _fill_const(ref, rows, d, val):
    vec = jnp.full((16,), val, jnp.float32)

    def row(i, carry):
        for jj in range(d // 16):
            ref[i, pl.ds(jj * 16, 16)] = vec
        return carry

    lax.fori_loop(0, rows, row, 0)


def _sc_degree(dst3, npad):
    nslab, C, ch = dst3.shape
    rpt = npad // 16
    mesh = plsc.VectorSubcoreMesh(core_axis_name="c", subcore_axis_name="s")

    def body(dst_hbm, out_hbm, didx, obuf, zbuf, acc):
        c = lax.axis_index("c")
        s = lax.axis_index("s")
        slab = c * 16 + s
        pltpu.sync_copy(dst_hbm.at[slab], didx)
        _fill_const(obuf, ch, 16, 1.0)
        _fill_const(zbuf, ch, 16, 0.0)
        for t in range(rpt // ch):
            pltpu.sync_copy(zbuf, acc.at[pl.ds(s * rpt + t * ch, ch)])
        plsc.subcore_barrier()

        def step(j, carry):
            pltpu.sync_copy(obuf, acc.at[didx.at[j]], add=True)
            return carry

        lax.fori_loop(0, C, step, 0)
        plsc.subcore_barrier()
        pltpu.sync_copy(acc.at[pl.ds(s * rpt, rpt)],
                        out_hbm.at[c, pl.ds(s * rpt, rpt)])

    f = pl.kernel(
        body,
        out_type=jax.ShapeDtypeStruct((2, npad, 16), jnp.float32),
        mesh=mesh,
        scratch_types=[
            pltpu.VMEM((C, ch), jnp.int32),
            pltpu.VMEM((ch, 16), jnp.float32),
            pltpu.VMEM((ch, 16), jnp.float32),
            pltpu.VMEM_SHARED((npad, 16), jnp.float32),
        ],
    )
    return f(dst3)


def _sc_agg(u, src3, dst3, npad):
    n, d = u.shape
    nslab, C, ch = src3.shape
    rpt = npad // 16
    mesh = plsc.VectorSubcoreMesh(core_axis_name="c", subcore_axis_name="s")

    def body(u_hbm, src_hbm, dst_hbm, out_hbm, sidx, didx, rows, zbuf, acc,
             gsem, ssem):
        c = lax.axis_index("c")
        s = lax.axis_index("s")
        slab = c * 16 + s
        pltpu.sync_copy(src_hbm.at[slab], sidx)
        pltpu.sync_copy(dst_hbm.at[slab], didx)
        _fill_const(zbuf, ch, d, 0.0)
        for t in range(rpt // ch):
            pltpu.sync_copy(zbuf, acc.at[pl.ds(s * rpt + t * ch, ch)])
        plsc.subcore_barrier()

        def step(j, carry):
            pltpu.async_copy(u_hbm.at[sidx.at[j]], rows.at[0], gsem).wait()
            pltpu.async_copy(rows.at[0], acc.at[didx.at[j]], ssem,
                             add=True).wait()
            return carry

        lax.fori_loop(0, C, step, 0)
        plsc.subcore_barrier()
        pltpu.sync_copy(acc.at[pl.ds(s * rpt, rpt)],
                        out_hbm.at[c, pl.ds(s * rpt, rpt)])

    f = pl.kernel(
        body,
        out_type=jax.ShapeDtypeStruct((2, npad, d), jnp.float32),
        mesh=mesh,
        scratch_types=[
            pltpu.VMEM((C, ch), jnp.int32),
            pltpu.VMEM((C, ch), jnp.int32),
            pltpu.VMEM((2, ch, d), jnp.float32),
            pltpu.VMEM((ch, d), jnp.float32),
            pltpu.VMEM_SHARED((npad, d), jnp.float32),
            pltpu.SemaphoreType.DMA,
            pltpu.SemaphoreType.DMA,
        ],
    )
    return f(u, src3, dst3)


def _mm_body(x_ref, w_ref, o_ref):
    o_ref[...] = jnp.dot(x_ref[...], w_ref[...],
                         preferred_element_type=jnp.float32)


def _tc_matmul(x, w):
    n, k = x.shape
    m = w.shape[1]
    return pl.pallas_call(
        _mm_body,
        grid=(n // _RB,),
        in_specs=[
            pl.BlockSpec((_RB, k), lambda i: (i, 0)),
            pl.BlockSpec((k, m), lambda i: (0, 0)),
        ],
        out_specs=pl.BlockSpec((_RB, m), lambda i: (i, 0)),
        out_shape=jax.ShapeDtypeStruct((n, m), jnp.float32),
    )(x, w)


def _scale_body(h_ref, a_ref, b_ref, o_ref):
    dinv = lax.rsqrt(1.0 + a_ref[:, 0:1] + b_ref[:, 0:1])
    o_ref[...] = h_ref[...] * dinv


def _tc_scale(h, d0, d1):
    n, m = h.shape
    return pl.pallas_call(
        _scale_body,
        grid=(n // _RB,),
        in_specs=[
            pl.BlockSpec((_RB, m), lambda i: (i, 0)),
            pl.BlockSpec((_RB, 16), lambda i: (i, 0)),
            pl.BlockSpec((_RB, 16), lambda i: (i, 0)),
        ],
        out_specs=pl.BlockSpec((_RB, m), lambda i: (i, 0)),
        out_shape=jax.ShapeDtypeStruct((n, m), jnp.float32),
    )(h, d0, d1)


def _l2_body(p0_ref, p1_ref, u1_ref, a_ref, b_ref, b1_ref, w2_ref, o_ref):
    dinv = lax.rsqrt(1.0 + a_ref[:, 0:1] + b_ref[:, 0:1])
    t = dinv * (p0_ref[...] + p1_ref[...] + u1_ref[...]) + b1_ref[...]
    t = jnp.maximum(t, 0.0)
    o_ref[...] = dinv * jnp.dot(t, w2_ref[...],
                                preferred_element_type=jnp.float32)


def _tc_layer2(p0, p1, u1, d0, d1, b1, w2):
    n, m = u1.shape
    ncls = w2.shape[1]
    return pl.pallas_call(
        _l2_body,
        grid=(n // _RB,),
        in_specs=[
            pl.BlockSpec((_RB, m), lambda i: (i, 0)),
            pl.BlockSpec((_RB, m), lambda i: (i, 0)),
            pl.BlockSpec((_RB, m), lambda i: (i, 0)),
            pl.BlockSpec((_RB, 16), lambda i: (i, 0)),
            pl.BlockSpec((_RB, 16), lambda i: (i, 0)),
            pl.BlockSpec((1, m), lambda i: (0, 0)),
            pl.BlockSpec((m, ncls), lambda i: (0, 0)),
        ],
        out_specs=pl.BlockSpec((_RB, ncls), lambda i: (i, 0)),
        out_shape=jax.ShapeDtypeStruct((n, ncls), jnp.float32),
    )(p0, p1, u1, d0, d1, b1, w2)


def _fin_body(q0_ref, q1_ref, u2_ref, a_ref, b_ref, b2_ref, o_ref):
    dinv = lax.rsqrt(1.0 + a_ref[:, 0:1] + b_ref[:, 0:1])
    z = dinv * (q0_ref[...] + q1_ref[...] + u2_ref[...]) + b2_ref[...]
    z = z - jnp.max(z, axis=1, keepdims=True)
    e = jnp.exp(z)
    o_ref[...] = e / jnp.sum(e, axis=1, keepdims=True)


def _tc_final(q0, q1, u2, d0, d1, b2):
    n, ncls = u2.shape
    return pl.pallas_call(
        _fin_body,
        grid=(n // _RB,),
        in_specs=[
            pl.BlockSpec((_RB, ncls), lambda i: (i, 0)),
            pl.BlockSpec((_RB, ncls), lambda i: (i, 0)),
            pl.BlockSpec((_RB, ncls), lambda i: (i, 0)),
            pl.BlockSpec((_RB, 16), lambda i: (i, 0)),
            pl.BlockSpec((_RB, 16), lambda i: (i, 0)),
            pl.BlockSpec((1, ncls), lambda i: (0, 0)),
        ],
        out_specs=pl.BlockSpec((_RB, ncls), lambda i: (i, 0)),
        out_shape=jax.ShapeDtypeStruct((n, ncls), jnp.float32),
    )(q0, q1, u2, d0, d1, b2)


def kernel(x, edge_index, W1, b1, W2, b2):
    n, _ = x.shape
    e = edge_index.shape[1]

    C = _cdiv(e, _NSLAB * _CH)
    epad = _NSLAB * C * _CH
    npad = (n // (16 * _CH) + 1) * (16 * _CH)

    pad = epad - e
    src = edge_index[0]
    dst = edge_index[1]
    src3 = jnp.concatenate(
        [src, jnp.zeros((pad,), jnp.int32)]).reshape(_NSLAB, C, _CH)
    dst3 = jnp.concatenate(
        [dst, jnp.full((pad,), n, jnp.int32)]).reshape(_NSLAB, C, _CH)

    deg = _sc_degree(dst3, npad)
    d0 = deg[0, :n]
    d1 = deg[1, :n]

    h1 = _tc_matmul(x, W1)
    u1 = _tc_scale(h1, d0, d1)

    p = _sc_agg(u1, src3, dst3, npad)
    u2 = _tc_layer2(p[0, :n], p[1, :n], u1, d0, d1,
                    b1.reshape(1, -1), W2)

    q = _sc_agg(u2, src3, dst3, npad)
    return _tc_final(q[0, :n], q[1, :n], u2, d0, d1, b2.reshape(1, -1))


# speedup vs baseline: 17.1212x; 17.1212x over previous
import functools

import jax
import jax.numpy as jnp
from jax import lax
from jax.experimental import pallas as pl
from jax.experimental.pallas import tpu as pltpu
from jax.experimental.pallas import tpu_sc as plsc

_CH = 128
_NSLAB = 32
_RB = 2000


def _cdiv(a, b):
    return (a + b - 1) // b


def _fill_const(ref, rows, d, val):
    vec = jnp.full((16,), val, jnp.float32)

    def row(i, carry):
        for jj in range(d // 16):
            ref[i, pl.ds(jj * 16, 16)] = vec
        return carry

    lax.fori_loop(0, rows, row, 0)


def _sc_degree(dst3, npad):
    nslab, C, ch = dst3.shape
    rpt = npad // 16
    mesh = plsc.VectorSubcoreMesh(core_axis_name="c", subcore_axis_name="s")

    def body(dst_hbm, out_hbm, didx, obuf, zbuf, acc):
        c = lax.axis_index("c")
        s = lax.axis_index("s")
        slab = c * 16 + s
        pltpu.sync_copy(dst_hbm.at[slab], didx)
        _fill_const(obuf, ch, 16, 1.0)
        _fill_const(zbuf, ch, 16, 0.0)
        for t in range(rpt // ch):
            pltpu.sync_copy(zbuf, acc.at[pl.ds(s * rpt + t * ch, ch)])
        plsc.subcore_barrier()

        def step(j, carry):
            pltpu.sync_copy(obuf, acc.at[didx.at[j]], add=True)
            return carry

        lax.fori_loop(0, C, step, 0)
        plsc.subcore_barrier()
        pltpu.sync_copy(acc.at[pl.ds(s * rpt, rpt)],
                        out_hbm.at[c, pl.ds(s * rpt, rpt)])

    f = pl.kernel(
        body,
        out_type=jax.ShapeDtypeStruct((2, npad, 16), jnp.float32),
        mesh=mesh,
        compiler_params=pltpu.CompilerParams(use_tc_tiling_on_sc=False),
        scratch_types=[
            pltpu.VMEM((C, ch), jnp.int32),
            pltpu.VMEM((ch, 16), jnp.float32),
            pltpu.VMEM((ch, 16), jnp.float32),
            pltpu.VMEM_SHARED((npad, 16), jnp.float32),
        ],
    )
    return f(dst3)


def _sc_agg(u, src3, dst3, npad):
    n, d = u.shape
    nslab, C, ch = src3.shape
    rpt = npad // 16
    mesh = plsc.VectorSubcoreMesh(core_axis_name="c", subcore_axis_name="s")

    def body(u_hbm, src_hbm, dst_hbm, out_hbm, sidx2, didx2, rows, acc,
             isem):
        c = lax.axis_index("c")
        s = lax.axis_index("s")
        slab = c * 16 + s
        _fill_const(rows, ch, d, 0.0)
        for t in range(rpt // ch):
            pltpu.sync_copy(rows, acc.at[pl.ds(s * rpt + t * ch, ch)])
        plsc.subcore_barrier()

        pltpu.sync_copy(src_hbm.at[slab, 0], sidx2.at[0])
        pltpu.sync_copy(dst_hbm.at[slab, 0], didx2.at[0])

        def step(j, carry):
            b = j % 2
            @pl.when(j + 1 < C)
            def _():
                pltpu.async_copy(src_hbm.at[slab, j + 1], sidx2.at[1 - b],
                                 isem)
                pltpu.async_copy(dst_hbm.at[slab, j + 1], didx2.at[1 - b],
                                 isem)

            pltpu.sync_copy(u_hbm.at[sidx2.at[b]], rows)
            pltpu.sync_copy(rows, acc.at[didx2.at[b]], add=True)

            @pl.when(j + 1 < C)
            def _():
                pltpu.make_async_copy(src_hbm.at[slab, j + 1],
                                      sidx2.at[1 - b], isem).wait()
                pltpu.make_async_copy(dst_hbm.at[slab, j + 1],
                                      didx2.at[1 - b], isem).wait()

            return carry

        lax.fori_loop(0, C, step, 0)
        plsc.subcore_barrier()
        pltpu.sync_copy(acc.at[pl.ds(s * rpt, rpt)],
                        out_hbm.at[c, pl.ds(s * rpt, rpt)])

    f = pl.kernel(
        body,
        out_type=jax.ShapeDtypeStruct((2, npad, d), jnp.float32),
        mesh=mesh,
        compiler_params=pltpu.CompilerParams(use_tc_tiling_on_sc=False),
        scratch_types=[
            pltpu.VMEM((2, ch), jnp.int32),
            pltpu.VMEM((2, ch), jnp.int32),
            pltpu.VMEM((ch, d), jnp.float32),
            pltpu.VMEM_SHARED((npad, d), jnp.float32),
            pltpu.SemaphoreType.DMA,
        ],
    )
    return f(u, src3, dst3)


def _mm_body(x_ref, w_ref, o_ref):
    o_ref[...] = jnp.dot(x_ref[...], w_ref[...],
                         preferred_element_type=jnp.float32)


def _tc_matmul(x, w):
    n, k = x.shape
    m = w.shape[1]
    return pl.pallas_call(
        _mm_body,
        grid=(n // _RB,),
        in_specs=[
            pl.BlockSpec((_RB, k), lambda i: (i, 0)),
            pl.BlockSpec((k, m), lambda i: (0, 0)),
        ],
        out_specs=pl.BlockSpec((_RB, m), lambda i: (i, 0)),
        out_shape=jax.ShapeDtypeStruct((n, m), jnp.float32),
    )(x, w)


def _scale_body(h_ref, a_ref, b_ref, o_ref):
    dinv = lax.rsqrt(1.0 + a_ref[:, 0:1] + b_ref[:, 0:1])
    o_ref[...] = h_ref[...] * dinv


def _tc_scale(h, d0, d1):
    n, m = h.shape
    return pl.pallas_call(
        _scale_body,
        grid=(n // _RB,),
        in_specs=[
            pl.BlockSpec((_RB, m), lambda i: (i, 0)),
            pl.BlockSpec((_RB, 16), lambda i: (i, 0)),
            pl.BlockSpec((_RB, 16), lambda i: (i, 0)),
        ],
        out_specs=pl.BlockSpec((_RB, m), lambda i: (i, 0)),
        out_shape=jax.ShapeDtypeStruct((n, m), jnp.float32),
    )(h, d0, d1)


def _l2_body(p0_ref, p1_ref, u1_ref, a_ref, b_ref, b1_ref, w2_ref, o_ref):
    dinv = lax.rsqrt(1.0 + a_ref[:, 0:1] + b_ref[:, 0:1])
    t = dinv * (p0_ref[...] + p1_ref[...] + u1_ref[...]) + b1_ref[...]
    t = jnp.maximum(t, 0.0)
    o_ref[...] = dinv * jnp.dot(t, w2_ref[...],
                                preferred_element_type=jnp.float32)


def _tc_layer2(p0, p1, u1, d0, d1, b1, w2):
    n, m = u1.shape
    ncls = w2.shape[1]
    return pl.pallas_call(
        _l2_body,
        grid=(n // _RB,),
        in_specs=[
            pl.BlockSpec((_RB, m), lambda i: (i, 0)),
            pl.BlockSpec((_RB, m), lambda i: (i, 0)),
            pl.BlockSpec((_RB, m), lambda i: (i, 0)),
            pl.BlockSpec((_RB, 16), lambda i: (i, 0)),
            pl.BlockSpec((_RB, 16), lambda i: (i, 0)),
            pl.BlockSpec((1, m), lambda i: (0, 0)),
            pl.BlockSpec((m, ncls), lambda i: (0, 0)),
        ],
        out_specs=pl.BlockSpec((_RB, ncls), lambda i: (i, 0)),
        out_shape=jax.ShapeDtypeStruct((n, ncls), jnp.float32),
    )(p0, p1, u1, d0, d1, b1, w2)


def _fin_body(q0_ref, q1_ref, u2_ref, a_ref, b_ref, b2_ref, o_ref):
    dinv = lax.rsqrt(1.0 + a_ref[:, 0:1] + b_ref[:, 0:1])
    z = dinv * (q0_ref[...] + q1_ref[...] + u2_ref[...]) + b2_ref[...]
    z = z - jnp.max(z, axis=1, keepdims=True)
    e = jnp.exp(z)
    o_ref[...] = e / jnp.sum(e, axis=1, keepdims=True)


def _tc_final(q0, q1, u2, d0, d1, b2):
    n, ncls = u2.shape
    return pl.pallas_call(
        _fin_body,
        grid=(n // _RB,),
        in_specs=[
            pl.BlockSpec((_RB, ncls), lambda i: (i, 0)),
            pl.BlockSpec((_RB, ncls), lambda i: (i, 0)),
            pl.BlockSpec((_RB, ncls), lambda i: (i, 0)),
            pl.BlockSpec((_RB, 16), lambda i: (i, 0)),
            pl.BlockSpec((_RB, 16), lambda i: (i, 0)),
            pl.BlockSpec((1, ncls), lambda i: (0, 0)),
        ],
        out_specs=pl.BlockSpec((_RB, ncls), lambda i: (i, 0)),
        out_shape=jax.ShapeDtypeStruct((n, ncls), jnp.float32),
    )(q0, q1, u2, d0, d1, b2)


def kernel(x, edge_index, W1, b1, W2, b2):
    n, _ = x.shape
    e = edge_index.shape[1]

    C = _cdiv(e, _NSLAB * _CH)
    epad = _NSLAB * C * _CH
    npad = (n // (16 * _CH) + 1) * (16 * _CH)

    pad = epad - e
    src = edge_index[0]
    dst = edge_index[1]
    src3 = jnp.concatenate(
        [src, jnp.zeros((pad,), jnp.int32)]).reshape(_NSLAB, C, _CH)
    dst3 = jnp.concatenate(
        [dst, jnp.full((pad,), n, jnp.int32)]).reshape(_NSLAB, C, _CH)

    deg = _sc_degree(dst3, npad)
    d0 = deg[0, :n]
    d1 = deg[1, :n]

    h1 = _tc_matmul(x, W1)
    u1 = _tc_scale(h1, d0, d1)

    p = _sc_agg(u1, src3, dst3, npad)
    u2 = _tc_layer2(p[0, :n], p[1, :n], u1, d0, d1,
                    b1.reshape(1, -1), W2)

    q = _sc_agg(u2, src3, dst3, npad)
    return _tc_final(q[0, :n], q[1, :n], u2, d0, d1, b2.reshape(1, -1))
